# Initial kernel scaffold; baseline (speedup 1.0000x reference)
import jax
import jax.numpy as jnp
from jax.experimental import pallas as pl


def kernel(x, coords, Wq, bq, Wkv, bkv, Wo, bo, Wiq, biq, Wik, bik, head_w):
    raise NotImplementedError("write your pallas kernel here")


# speedup vs baseline: 50.3307x; 50.3307x over previous
import math

import jax
import jax.numpy as jnp
from jax.experimental import pallas as pl
from jax.experimental.pallas import tpu as pltpu

B, T, D = 2, 2048, 1024
H, DH = 16, 64
HI, DI = 4, 64
TOPK = 512
THETA = 10000.0
COORD_HIGH = 100000.0

BQ = 256
NBLK = (B * T) // BQ
QB = T // BQ

HIGH = jax.lax.Precision.HIGHEST
NEG = -1e9


def _rope(xx, cx, cy):
    rows, w = xx.shape
    l = jax.lax.broadcasted_iota(jnp.int32, (rows, w), 1) % DH
    i16 = (l % 32) // 2
    invf = jnp.exp(i16.astype(jnp.float32) * (-math.log(THETA) / 16.0))
    c = jnp.where(l < 32, cx, cy)
    freqs = c * invf
    cos = jnp.cos(freqs)
    sin = jnp.sin(freqs)
    even = (l % 2) == 0
    left = pltpu.roll(xx, w - 1, 1)
    right = pltpu.roll(xx, 1, 1)
    rot = jnp.where(even, -left, right)
    return xx * cos + rot * sin


def _proj_kernel(x_ref, coords_ref, wq_ref, bq_ref, wkv_ref, bkv_ref,
                 wiq_ref, biq_ref, wik_ref, bik_ref,
                 q_ref, k_ref, v_ref, qp_ref, kp_ref):
    x = x_ref[...]
    q = jnp.dot(x, wq_ref[...]) + bq_ref[...]
    kv = jnp.dot(x, wkv_ref[...]) + bkv_ref[...]
    qp_ref[...] = jnp.dot(x, wiq_ref[...]) + biq_ref[...]
    kp_ref[...] = jnp.dot(x, wik_ref[...]) + bik_ref[...]
    cf = coords_ref[...].astype(jnp.float32) / COORD_HIGH
    cx = cf[:, 0:1]
    cy = cf[:, 1:2]
    q_ref[...] = _rope(q, cx, cy)
    k_ref[...] = _rope(kv[:, :DH], cx, cy)
    v_ref[...] = kv[:, DH:]


def _excl_cumsum_lanes(x, col):
    s = x
    sh = 1
    while sh < T:
        r = pltpu.roll(s, sh, 1)
        s = s + jnp.where(col >= sh, r, 0)
        sh *= 2
    return s - x


def _attn_kernel(qp_ref, kp_ref, q_ref, k_ref, v_ref, hw_ref, wo_ref, bo_ref,
                 out_ref, acc_ref):
    qb = pl.program_id(1)

    qp = qp_ref[...]
    kp = kp_ref[...]
    sc = jnp.zeros((BQ, T), jnp.float32)
    for h in range(HI - 1, -1, -1):
        lg = jax.lax.dot_general(qp[:, h * DI:(h + 1) * DI],
                                 kp[:, h * DI:(h + 1) * DI],
                                 (((1,), (1,)), ((), ())))
        lgb = jnp.maximum(lg, 0.0).astype(jnp.bfloat16).astype(jnp.float32)
        sc = sc + hw_ref[h] * lgb
    row = qb * BQ + jax.lax.broadcasted_iota(jnp.int32, (BQ, T), 0)
    col = jax.lax.broadcasted_iota(jnp.int32, (BQ, T), 1)
    sc = jnp.where(col <= row, sc, NEG)

    ki = jax.lax.bitcast_convert_type(sc, jnp.int32)
    key = ki ^ ((ki >> 31) & jnp.int32(0x7FFFFFFF))
    cnt_hi = jnp.sum((key >= 0).astype(jnp.int32), axis=1, keepdims=True)
    base0 = jnp.where(cnt_hi >= TOPK, jnp.int32(0), jnp.int32(-2147483648))

    def bs_body(i, base):
        cand = base + (jnp.int32(1) << (jnp.int32(30) - i))
        cnt = jnp.sum((key >= cand).astype(jnp.int32), axis=1, keepdims=True)
        return jnp.where(cnt >= TOPK, cand, base)

    thr = jax.lax.fori_loop(0, 31, bs_body, base0)

    gt = key > thr
    tie = key == thr
    tie_i = tie.astype(jnp.int32)
    c_gt = jnp.sum(gt.astype(jnp.int32), axis=1, keepdims=True)
    need = TOPK - c_gt
    tie_rank = _excl_cumsum_lanes(tie_i, col)
    sel = gt | (tie & (tie_rank < need))
    attend = sel & (sc > -1e8)
    amask = jnp.where(attend, 0.0, NEG)

    def _rnd(z):
        return z.astype(jnp.bfloat16).astype(jnp.float32)

    k = _rnd(k_ref[...])
    v = _rnd(v_ref[...])
    qblk = q_ref[...]
    acc_ref[...] = jnp.zeros((BQ, D), jnp.float32)
    for h in range(H):
        qh = _rnd(qblk[:, h * DH:(h + 1) * DH])
        al = jax.lax.dot_general(qh, k, (((1,), (1,)), ((), ()))
                                 ) * (1.0 / 8.0) + amask
        m = jnp.max(al, axis=1, keepdims=True)
        e = jnp.exp(al - m)
        p = e / jnp.sum(e, axis=1, keepdims=True)
        oh = jnp.dot(_rnd(p), v)
        acc_ref[...] += jnp.dot(oh, wo_ref[h * DH:(h + 1) * DH, :])
    out_ref[...] = acc_ref[...] + bo_ref[...]


def _round_bf16_bits(x):
    i = jax.lax.bitcast_convert_type(x, jnp.int32)
    i = (i + jnp.int32(0x7FFF) + ((i >> 16) & jnp.int32(1))) & jnp.int32(-65536)
    return jax.lax.bitcast_convert_type(i, jnp.float32)


def kernel(x, coords, Wq, bq, Wkv, bkv, Wo, bo, Wiq, biq, Wik, bik, head_w):
    xf = x.reshape(B * T, D)
    cf = coords.reshape(B * T, 2)

    q, k, v, qp, kp = pl.pallas_call(
        _proj_kernel,
        grid=(NBLK,),
        in_specs=[
            pl.BlockSpec((BQ, D), lambda i: (i, 0)),
            pl.BlockSpec((BQ, 2), lambda i: (i, 0)),
            pl.BlockSpec((D, D), lambda i: (0, 0)),
            pl.BlockSpec((1, D), lambda i: (0, 0)),
            pl.BlockSpec((D, 2 * DH), lambda i: (0, 0)),
            pl.BlockSpec((1, 2 * DH), lambda i: (0, 0)),
            pl.BlockSpec((D, HI * DI), lambda i: (0, 0)),
            pl.BlockSpec((1, HI * DI), lambda i: (0, 0)),
            pl.BlockSpec((D, HI * DI), lambda i: (0, 0)),
            pl.BlockSpec((1, HI * DI), lambda i: (0, 0)),
        ],
        out_specs=[
            pl.BlockSpec((BQ, D), lambda i: (i, 0)),
            pl.BlockSpec((BQ, DH), lambda i: (i, 0)),
            pl.BlockSpec((BQ, DH), lambda i: (i, 0)),
            pl.BlockSpec((BQ, HI * DI), lambda i: (i, 0)),
            pl.BlockSpec((BQ, HI * DI), lambda i: (i, 0)),
        ],
        out_shape=[
            jax.ShapeDtypeStruct((B * T, D), jnp.float32),
            jax.ShapeDtypeStruct((B * T, DH), jnp.float32),
            jax.ShapeDtypeStruct((B * T, DH), jnp.float32),
            jax.ShapeDtypeStruct((B * T, HI * DI), jnp.float32),
            jax.ShapeDtypeStruct((B * T, HI * DI), jnp.float32),
        ],
    )(xf, cf, Wq, bq.reshape(1, D), Wkv, bkv.reshape(1, 2 * DH),
      Wiq, biq.reshape(1, HI * DI), Wik, bik.reshape(1, HI * DI))

    out = pl.pallas_call(
        _attn_kernel,
        grid=(B, QB),
        in_specs=[
            pl.BlockSpec((BQ, HI * DI), lambda b, j: (b * QB + j, 0)),
            pl.BlockSpec((T, HI * DI), lambda b, j: (b, 0)),
            pl.BlockSpec((BQ, D), lambda b, j: (b * QB + j, 0)),
            pl.BlockSpec((T, DH), lambda b, j: (b, 0)),
            pl.BlockSpec((T, DH), lambda b, j: (b, 0)),
            pl.BlockSpec(memory_space=pltpu.SMEM),
            pl.BlockSpec((D, D), lambda b, j: (0, 0)),
            pl.BlockSpec((1, D), lambda b, j: (0, 0)),
        ],
        out_specs=pl.BlockSpec((BQ, D), lambda b, j: (b * QB + j, 0)),
        out_shape=jax.ShapeDtypeStruct((B * T, D), jnp.float32),
        scratch_shapes=[pltpu.VMEM((BQ, D), jnp.float32)],
    )(qp, kp, q, k, v, _round_bf16_bits(head_w), Wo, bo.reshape(1, D))

    return out.reshape(B, T, D)


# speedup vs baseline: 54.5274x; 1.0834x over previous
import math

import jax
import jax.numpy as jnp
from jax.experimental import pallas as pl
from jax.experimental.pallas import tpu as pltpu

B, T, D = 2, 2048, 1024
H, DH = 16, 64
HI, DI = 4, 64
TOPK = 512
THETA = 10000.0
COORD_HIGH = 100000.0

BQ = 256
NBLK = (B * T) // BQ
QB = T // BQ

HIGH = jax.lax.Precision.HIGHEST
NEG = -1e9


def _rope(xx, cx, cy):
    rows, w = xx.shape
    l = jax.lax.broadcasted_iota(jnp.int32, (rows, w), 1) % DH
    i16 = (l % 32) // 2
    invf = jnp.exp(i16.astype(jnp.float32) * (-math.log(THETA) / 16.0))
    c = jnp.where(l < 32, cx, cy)
    freqs = c * invf
    cos = jnp.cos(freqs)
    sin = jnp.sin(freqs)
    even = (l % 2) == 0
    left = pltpu.roll(xx, w - 1, 1)
    right = pltpu.roll(xx, 1, 1)
    rot = jnp.where(even, -left, right)
    return xx * cos + rot * sin


def _proj_kernel(x_ref, coords_ref, wq_ref, bq_ref, wkv_ref, bkv_ref,
                 wiq_ref, biq_ref, wik_ref, bik_ref,
                 q_ref, k_ref, v_ref, qp_ref, kp_ref):
    x = x_ref[...]
    q = jnp.dot(x, wq_ref[...]) + bq_ref[...]
    kv = jnp.dot(x, wkv_ref[...]) + bkv_ref[...]
    qp_ref[...] = jnp.dot(x, wiq_ref[...]) + biq_ref[...]
    kp_ref[...] = jnp.dot(x, wik_ref[...]) + bik_ref[...]
    cf = coords_ref[...].astype(jnp.float32) / COORD_HIGH
    cx = cf[:, 0:1]
    cy = cf[:, 1:2]
    q_ref[...] = _rope(q, cx, cy)
    k_ref[...] = _rope(kv[:, :DH], cx, cy)
    v_ref[...] = kv[:, DH:]


def _excl_cumsum_lanes(x, col):
    s = x
    sh = 1
    while sh < T:
        r = pltpu.roll(s, sh, 1)
        s = s + jnp.where(col >= sh, r, 0)
        sh *= 2
    return s - x


def _attn_kernel(qp_ref, kp_ref, q_ref, k_ref, v_ref, hw_ref, wo_ref, bo_ref,
                 out_ref, acc_ref, amask_ref):
    qb = pl.program_id(1)
    row = qb * BQ + jax.lax.broadcasted_iota(jnp.int32, (BQ, T), 0)
    col = jax.lax.broadcasted_iota(jnp.int32, (BQ, T), 1)

    @pl.when(qb * BQ + BQ <= TOPK)
    def _():
        amask_ref[...] = jnp.where(col <= row, 0.0, NEG)

    @pl.when(qb * BQ + BQ > TOPK)
    def _():
        qp = qp_ref[...]
        kp = kp_ref[...]
        sc = jnp.zeros((BQ, T), jnp.float32)
        for h in range(HI - 1, -1, -1):
            lg = jax.lax.dot_general(qp[:, h * DI:(h + 1) * DI],
                                     kp[:, h * DI:(h + 1) * DI],
                                     (((1,), (1,)), ((), ())))
            lgb = jnp.maximum(lg, 0.0).astype(jnp.bfloat16).astype(jnp.float32)
            sc = sc + hw_ref[h] * lgb
        sc = jnp.where(col <= row, sc, NEG)

        ki = jax.lax.bitcast_convert_type(sc, jnp.int32)
        key = ki ^ ((ki >> 31) & jnp.int32(0x7FFFFFFF))
        cnt_hi = jnp.sum((key >= 0).astype(jnp.int32), axis=1, keepdims=True)
        base0 = jnp.where(cnt_hi >= TOPK, jnp.int32(0), jnp.int32(-2147483648))

        def bs_body(i, base):
            cand = base + (jnp.int32(1) << (jnp.int32(30) - i))
            cnt = jnp.sum((key >= cand).astype(jnp.int32), axis=1, keepdims=True)
            return jnp.where(cnt >= TOPK, cand, base)

        thr = jax.lax.fori_loop(0, 31, bs_body, base0)

        gt = key > thr
        tie = key == thr
        c_gt = jnp.sum(gt.astype(jnp.int32), axis=1, keepdims=True)
        need = TOPK - c_gt
        tie_rank = _excl_cumsum_lanes(tie.astype(jnp.int32), col)
        sel = gt | (tie & (tie_rank < need))
        attend = sel & (sc > -1e8)
        amask_ref[...] = jnp.where(attend, 0.0, NEG)

    amask = amask_ref[...]
    k = k_ref[...]
    v = v_ref[...]
    qblk = q_ref[...]
    acc_ref[...] = jnp.zeros((BQ, D), jnp.float32)
    for h in range(H):
        qh = qblk[:, h * DH:(h + 1) * DH]
        al = jax.lax.dot_general(qh, k, (((1,), (1,)), ((), ()))
                                 ) * (1.0 / 8.0) + amask
        m = jnp.max(al, axis=1, keepdims=True)
        e = jnp.exp(al - m)
        p = e / jnp.sum(e, axis=1, keepdims=True)
        oh = jnp.dot(p, v)
        acc_ref[...] += jnp.dot(oh, wo_ref[h * DH:(h + 1) * DH, :])
    out_ref[...] = acc_ref[...] + bo_ref[...]


def _round_bf16_bits(x):
    i = jax.lax.bitcast_convert_type(x, jnp.int32)
    i = (i + jnp.int32(0x7FFF) + ((i >> 16) & jnp.int32(1))) & jnp.int32(-65536)
    return jax.lax.bitcast_convert_type(i, jnp.float32)


def kernel(x, coords, Wq, bq, Wkv, bkv, Wo, bo, Wiq, biq, Wik, bik, head_w):
    xf = x.reshape(B * T, D)
    cf = coords.reshape(B * T, 2)

    q, k, v, qp, kp = pl.pallas_call(
        _proj_kernel,
        grid=(NBLK,),
        in_specs=[
            pl.BlockSpec((BQ, D), lambda i: (i, 0)),
            pl.BlockSpec((BQ, 2), lambda i: (i, 0)),
            pl.BlockSpec((D, D), lambda i: (0, 0)),
            pl.BlockSpec((1, D), lambda i: (0, 0)),
            pl.BlockSpec((D, 2 * DH), lambda i: (0, 0)),
            pl.BlockSpec((1, 2 * DH), lambda i: (0, 0)),
            pl.BlockSpec((D, HI * DI), lambda i: (0, 0)),
            pl.BlockSpec((1, HI * DI), lambda i: (0, 0)),
            pl.BlockSpec((D, HI * DI), lambda i: (0, 0)),
            pl.BlockSpec((1, HI * DI), lambda i: (0, 0)),
        ],
        out_specs=[
            pl.BlockSpec((BQ, D), lambda i: (i, 0)),
            pl.BlockSpec((BQ, DH), lambda i: (i, 0)),
            pl.BlockSpec((BQ, DH), lambda i: (i, 0)),
            pl.BlockSpec((BQ, HI * DI), lambda i: (i, 0)),
            pl.BlockSpec((BQ, HI * DI), lambda i: (i, 0)),
        ],
        out_shape=[
            jax.ShapeDtypeStruct((B * T, D), jnp.float32),
            jax.ShapeDtypeStruct((B * T, DH), jnp.float32),
            jax.ShapeDtypeStruct((B * T, DH), jnp.float32),
            jax.ShapeDtypeStruct((B * T, HI * DI), jnp.float32),
            jax.ShapeDtypeStruct((B * T, HI * DI), jnp.float32),
        ],
    )(xf, cf, Wq, bq.reshape(1, D), Wkv, bkv.reshape(1, 2 * DH),
      Wiq, biq.reshape(1, HI * DI), Wik, bik.reshape(1, HI * DI))

    out = pl.pallas_call(
        _attn_kernel,
        grid=(B, QB),
        in_specs=[
            pl.BlockSpec((BQ, HI * DI), lambda b, j: (b * QB + j, 0)),
            pl.BlockSpec((T, HI * DI), lambda b, j: (b, 0)),
            pl.BlockSpec((BQ, D), lambda b, j: (b * QB + j, 0)),
            pl.BlockSpec((T, DH), lambda b, j: (b, 0)),
            pl.BlockSpec((T, DH), lambda b, j: (b, 0)),
            pl.BlockSpec(memory_space=pltpu.SMEM),
            pl.BlockSpec((D, D), lambda b, j: (0, 0)),
            pl.BlockSpec((1, D), lambda b, j: (0, 0)),
        ],
        out_specs=pl.BlockSpec((BQ, D), lambda b, j: (b * QB + j, 0)),
        out_shape=jax.ShapeDtypeStruct((B * T, D), jnp.float32),
        scratch_shapes=[pltpu.VMEM((BQ, D), jnp.float32),
                        pltpu.VMEM((BQ, T), jnp.float32)],
    )(qp, kp, q, k, v, _round_bf16_bits(head_w), Wo, bo.reshape(1, D))

    return out.reshape(B, T, D)


# speedup vs baseline: 58.5812x; 1.0743x over previous
import math

import jax
import jax.numpy as jnp
from jax.experimental import pallas as pl
from jax.experimental.pallas import tpu as pltpu

B, T, D = 2, 2048, 1024
H, DH = 16, 64
HI, DI = 4, 64
TOPK = 512
THETA = 10000.0
COORD_HIGH = 100000.0

BQ = 256
NBLK = (B * T) // BQ
QB = T // BQ

HIGH = jax.lax.Precision.HIGHEST
NEG = -1e9


def _rope(xx, cx, cy):
    rows, w = xx.shape
    l = jax.lax.broadcasted_iota(jnp.int32, (rows, w), 1) % DH
    i16 = (l % 32) // 2
    invf = jnp.exp(i16.astype(jnp.float32) * (-math.log(THETA) / 16.0))
    c = jnp.where(l < 32, cx, cy)
    freqs = c * invf
    cos = jnp.cos(freqs)
    sin = jnp.sin(freqs)
    even = (l % 2) == 0
    left = pltpu.roll(xx, w - 1, 1)
    right = pltpu.roll(xx, 1, 1)
    rot = jnp.where(even, -left, right)
    return xx * cos + rot * sin


def _proj_kernel(x_ref, coords_ref, wq_ref, bq_ref, wkv_ref, bkv_ref,
                 wiq_ref, biq_ref, wik_ref, bik_ref,
                 q_ref, k_ref, v_ref, qp_ref, kp_ref):
    x = x_ref[...]
    q = jnp.dot(x, wq_ref[...]) + bq_ref[...]
    kv = jnp.dot(x, wkv_ref[...]) + bkv_ref[...]
    qp_ref[...] = jnp.dot(x, wiq_ref[...]) + biq_ref[...]
    kp_ref[...] = jnp.dot(x, wik_ref[...]) + bik_ref[...]
    cf = coords_ref[...].astype(jnp.float32) / COORD_HIGH
    cx = cf[:, 0:1]
    cy = cf[:, 1:2]
    q_ref[...] = _rope(q, cx, cy)
    k_ref[...] = _rope(kv[:, :DH], cx, cy)
    v_ref[...] = kv[:, DH:]


def _excl_cumsum_lanes(x, col):
    s = x
    sh = 1
    while sh < T:
        r = pltpu.roll(s, sh, 1)
        s = s + jnp.where(col >= sh, r, 0)
        sh *= 2
    return s - x


def _attn_kernel(qp_ref, kp_ref, q_ref, k_ref, v_ref, hw_ref, wo_ref, bo_ref,
                 out_ref, acc_ref, amask_ref):
    qb = pl.program_id(1)
    row = qb * BQ + jax.lax.broadcasted_iota(jnp.int32, (BQ, T), 0)
    col = jax.lax.broadcasted_iota(jnp.int32, (BQ, T), 1)

    @pl.when(qb * BQ + BQ <= TOPK)
    def _():
        amask_ref[...] = jnp.where(col <= row, 0.0, NEG)

    @pl.when(qb * BQ + BQ > TOPK)
    def _():
        qp = qp_ref[...]
        kp = kp_ref[...]
        sc = jnp.zeros((BQ, T), jnp.float32)
        for h in range(HI - 1, -1, -1):
            lg = jax.lax.dot_general(qp[:, h * DI:(h + 1) * DI],
                                     kp[:, h * DI:(h + 1) * DI],
                                     (((1,), (1,)), ((), ())))
            lgb = jnp.maximum(lg, 0.0).astype(jnp.bfloat16).astype(jnp.float32)
            sc = sc + hw_ref[h] * lgb
        sc = jnp.where(col <= row, sc, NEG)

        ki = jax.lax.bitcast_convert_type(sc, jnp.int32)
        key = ki ^ ((ki >> 31) & jnp.int32(0x7FFFFFFF))
        cnt_hi = jnp.sum((key >= 0).astype(jnp.int32), axis=1, keepdims=True)
        base0 = jnp.where(cnt_hi >= TOPK, jnp.int32(0), jnp.int32(-2147483648))

        def bs_body(i, base):
            cand = base + (jnp.int32(1) << (jnp.int32(30) - i))
            cnt = jnp.sum((key >= cand).astype(jnp.int32), axis=1, keepdims=True)
            return jnp.where(cnt >= TOPK, cand, base)

        thr = jax.lax.fori_loop(0, 31, bs_body, base0)

        gt = key > thr
        tie = key == thr
        c_gt = jnp.sum(gt.astype(jnp.int32), axis=1, keepdims=True)
        need = TOPK - c_gt
        tie_rank = _excl_cumsum_lanes(tie.astype(jnp.int32), col)
        sel = gt | (tie & (tie_rank < need))
        attend = sel & (sc > -1e8)
        amask_ref[...] = jnp.where(attend, 0.0, NEG)

    amask = amask_ref[...]
    k = k_ref[...]
    v = v_ref[...]
    qblk = q_ref[...] * (1.0 / 8.0)
    acc_ref[...] = jnp.zeros((BQ, D), jnp.float32)
    for h in range(H):
        qh = qblk[:, h * DH:(h + 1) * DH]
        al = jax.lax.dot_general(qh, k, (((1,), (1,)), ((), ()))) + amask
        m = jnp.max(al, axis=1, keepdims=True)
        e = jnp.exp(al - m)
        oh = jnp.dot(e, v) / jnp.sum(e, axis=1, keepdims=True)
        acc_ref[...] += jnp.dot(oh, wo_ref[h * DH:(h + 1) * DH, :])
    out_ref[...] = acc_ref[...] + bo_ref[...]


def _round_bf16_bits(x):
    i = jax.lax.bitcast_convert_type(x, jnp.int32)
    i = (i + jnp.int32(0x7FFF) + ((i >> 16) & jnp.int32(1))) & jnp.int32(-65536)
    return jax.lax.bitcast_convert_type(i, jnp.float32)


def kernel(x, coords, Wq, bq, Wkv, bkv, Wo, bo, Wiq, biq, Wik, bik, head_w):
    xf = x.reshape(B * T, D)
    cf = coords.reshape(B * T, 2)

    q, k, v, qp, kp = pl.pallas_call(
        _proj_kernel,
        grid=(NBLK,),
        in_specs=[
            pl.BlockSpec((BQ, D), lambda i: (i, 0)),
            pl.BlockSpec((BQ, 2), lambda i: (i, 0)),
            pl.BlockSpec((D, D), lambda i: (0, 0)),
            pl.BlockSpec((1, D), lambda i: (0, 0)),
            pl.BlockSpec((D, 2 * DH), lambda i: (0, 0)),
            pl.BlockSpec((1, 2 * DH), lambda i: (0, 0)),
            pl.BlockSpec((D, HI * DI), lambda i: (0, 0)),
            pl.BlockSpec((1, HI * DI), lambda i: (0, 0)),
            pl.BlockSpec((D, HI * DI), lambda i: (0, 0)),
            pl.BlockSpec((1, HI * DI), lambda i: (0, 0)),
        ],
        out_specs=[
            pl.BlockSpec((BQ, D), lambda i: (i, 0)),
            pl.BlockSpec((BQ, DH), lambda i: (i, 0)),
            pl.BlockSpec((BQ, DH), lambda i: (i, 0)),
            pl.BlockSpec((BQ, HI * DI), lambda i: (i, 0)),
            pl.BlockSpec((BQ, HI * DI), lambda i: (i, 0)),
        ],
        out_shape=[
            jax.ShapeDtypeStruct((B * T, D), jnp.float32),
            jax.ShapeDtypeStruct((B * T, DH), jnp.float32),
            jax.ShapeDtypeStruct((B * T, DH), jnp.float32),
            jax.ShapeDtypeStruct((B * T, HI * DI), jnp.float32),
            jax.ShapeDtypeStruct((B * T, HI * DI), jnp.float32),
        ],
    )(xf, cf, Wq, bq.reshape(1, D), Wkv, bkv.reshape(1, 2 * DH),
      Wiq, biq.reshape(1, HI * DI), Wik, bik.reshape(1, HI * DI))

    out = pl.pallas_call(
        _attn_kernel,
        grid=(B, QB),
        in_specs=[
            pl.BlockSpec((BQ, HI * DI), lambda b, j: (b * QB + j, 0)),
            pl.BlockSpec((T, HI * DI), lambda b, j: (b, 0)),
            pl.BlockSpec((BQ, D), lambda b, j: (b * QB + j, 0)),
            pl.BlockSpec((T, DH), lambda b, j: (b, 0)),
            pl.BlockSpec((T, DH), lambda b, j: (b, 0)),
            pl.BlockSpec(memory_space=pltpu.SMEM),
            pl.BlockSpec((D, D), lambda b, j: (0, 0)),
            pl.BlockSpec((1, D), lambda b, j: (0, 0)),
        ],
        out_specs=pl.BlockSpec((BQ, D), lambda b, j: (b * QB + j, 0)),
        out_shape=jax.ShapeDtypeStruct((B * T, D), jnp.float32),
        scratch_shapes=[pltpu.VMEM((BQ, D), jnp.float32),
                        pltpu.VMEM((BQ, T), jnp.float32)],
    )(qp, kp, q, k, v, _round_bf16_bits(head_w), Wo, bo.reshape(1, D))

    return out.reshape(B, T, D)
